# baseline (speedup 1.0000x reference)
import functools

import jax
import jax.numpy as jnp
from jax import lax
from jax.experimental import pallas as pl
from jax.experimental.pallas import tpu as pltpu
from jax.experimental.pallas import tpu_sc as plsc

B = 1024
S = 200
DIM = 64
NC = 2
NS = 16
NW = NC * NS
ROWS = B * S
ROWS_PER_W = ROWS // NW
SEQS_PER_W = ROWS_PER_W // S
LANES = 16
VECS_PER_ROW = DIM // LANES


def _body(x_hbm, emb_hbm, pos_hbm, out_hbm, idx_v, rows_v, pos_v, sem):
    cid = lax.axis_index("c")
    sid = lax.axis_index("s")
    wid = sid * NC + cid
    base = wid * ROWS_PER_W

    pltpu.sync_copy(pos_hbm.at[pl.ds(0, S)], pos_v)

    def seq_body(k, carry):
        row0 = base + k * S
        pltpu.sync_copy(x_hbm.at[pl.ds(row0, S)], idx_v)
        pltpu.async_copy(emb_hbm.at[idx_v], rows_v, sem).wait()

        def add_body(i, carry2):
            for j in range(VECS_PER_ROW):
                sl = pl.ds(j * LANES, LANES)
                rows_v[i, sl] = rows_v[i, sl] + pos_v[i, sl]
            return carry2

        lax.fori_loop(0, S, add_body, 0)
        pltpu.sync_copy(rows_v, out_hbm.at[pl.ds(row0, S)])
        return carry

    lax.fori_loop(0, SEQS_PER_W, seq_body, 0)


@jax.jit
def kernel(x, emb, pos_emb):
    x_flat = x.reshape(ROWS)
    mesh = plsc.VectorSubcoreMesh(core_axis_name="c", subcore_axis_name="s")
    out = pl.kernel(
        _body,
        out_type=jax.ShapeDtypeStruct((ROWS, DIM), jnp.float32),
        mesh=mesh,
        scratch_types=[
            pltpu.VMEM((S,), jnp.int32),
            pltpu.VMEM((S, DIM), jnp.float32),
            pltpu.VMEM((S, DIM), jnp.float32),
            pltpu.SemaphoreType.DMA,
        ],
        compiler_params=pltpu.CompilerParams(use_tc_tiling_on_sc=False),
    )(x_flat, emb, pos_emb)
    return out.reshape(B, S, DIM)


# speedup vs baseline: 1.0674x; 1.0674x over previous
import jax
import jax.numpy as jnp
from jax import lax
from jax.experimental import pallas as pl
from jax.experimental.pallas import tpu as pltpu
from jax.experimental.pallas import tpu_sc as plsc

B = 1024
S = 200
DIM = 64
NC = 2
NS = 16
NW = NC * NS
ROWS = B * S
ROWS_PER_W = ROWS // NW
CH = 2 * S
NCH = ROWS_PER_W // CH
LANES = 16
VECS_PER_ROW = DIM // LANES


def _body(x_hbm, emb_hbm, pos_hbm, out_hbm,
          idx0, idx1, rows0, rows1, pos_v,
          isem0, isem1, gsem0, gsem1, osem0, osem1):
    idx_v = (idx0, idx1)
    rows_v = (rows0, rows1)
    isem = (isem0, isem1)
    gsem = (gsem0, gsem1)
    osem = (osem0, osem1)

    wid = lax.axis_index("s") * NC + lax.axis_index("c")
    base = wid * ROWS_PER_W

    pltpu.sync_copy(pos_hbm.at[pl.ds(0, S)], pos_v)

    def start_idx(g, sl):
        row0 = base + g * CH
        pltpu.make_async_copy(
            x_hbm.at[pl.ds(row0, CH)], idx_v[sl], isem[sl]).start()

    def wait_idx(sl):
        pltpu.make_async_copy(
            x_hbm.at[pl.ds(base, CH)], idx_v[sl], isem[sl]).wait()

    def start_gather(sl):
        pltpu.make_async_copy(
            emb_hbm.at[idx_v[sl]], rows_v[sl], gsem[sl]).start()

    def wait_gather(sl):
        pltpu.make_async_copy(
            emb_hbm.at[idx_v[sl]], rows_v[sl], gsem[sl]).wait()

    def start_out(g, sl):
        row0 = base + g * CH
        pltpu.make_async_copy(
            rows_v[sl], out_hbm.at[pl.ds(row0, CH)], osem[sl]).start()

    def wait_out(sl):
        pltpu.make_async_copy(
            rows_v[sl], out_hbm.at[pl.ds(base, CH)], osem[sl]).wait()

    def add_pos(sl):
        r = rows_v[sl]

        @plsc.parallel_loop(0, S, step=1, unroll=8)
        def _(i):
            for j in range(VECS_PER_ROW):
                c = pl.ds(j * LANES, LANES)
                p = pos_v[i, c]
                r[i, c] = r[i, c] + p
                r[i + S, c] = r[i + S, c] + p

    start_idx(0, 0)
    start_idx(1, 1)
    wait_idx(0)
    start_gather(0)

    for g in range(NCH):
        sl = g & 1
        nsl = sl ^ 1
        if g + 1 < NCH:
            wait_idx(nsl)
            if g >= 1:
                wait_out(nsl)
            start_gather(nsl)
        wait_gather(sl)
        if g + 2 < NCH:
            start_idx(g + 2, sl)
        add_pos(sl)
        start_out(g, sl)

    wait_out(0)
    wait_out(1)


@jax.jit
def kernel(x, emb, pos_emb):
    x_flat = x.reshape(ROWS)
    mesh = plsc.VectorSubcoreMesh(core_axis_name="c", subcore_axis_name="s")
    out = pl.kernel(
        _body,
        out_type=jax.ShapeDtypeStruct((ROWS, DIM), jnp.float32),
        mesh=mesh,
        scratch_types=[
            pltpu.VMEM((CH,), jnp.int32),
            pltpu.VMEM((CH,), jnp.int32),
            pltpu.VMEM((CH, DIM), jnp.float32),
            pltpu.VMEM((CH, DIM), jnp.float32),
            pltpu.VMEM((S, DIM), jnp.float32),
            pltpu.SemaphoreType.DMA,
            pltpu.SemaphoreType.DMA,
            pltpu.SemaphoreType.DMA,
            pltpu.SemaphoreType.DMA,
            pltpu.SemaphoreType.DMA,
            pltpu.SemaphoreType.DMA,
        ],
        compiler_params=pltpu.CompilerParams(use_tc_tiling_on_sc=False),
    )(x_flat, emb, pos_emb)
    return out.reshape(B, S, DIM)
